# baseline (speedup 1.0000x reference)
import functools

import jax
import jax.numpy as jnp
from jax import lax
from jax.experimental import pallas as pl
from jax.experimental.pallas import tpu as pltpu
from jax.experimental.pallas import tpu_sc as plsc

BATCH = 16384
K = 1000
BR = 512

L = 16
NC = 2
NS = 16
NW = NC * NS
CHUNK1 = BATCH // NS
CHUNK2 = BATCH // NW
KPAD = 1024
SW = 128
R1 = CHUNK1 // SW

RS = 160
NSC = NW * RS
NT = BATCH - NSC
NBUF = 4
NVF = (K - L) // L
TAIL = K - L
NBT = NT // BR


def _sc_body(lbl_hbm, c_hbm, cr_hbm, m_hbm, s_hbm, g_hbm, d_hbm,
             lbl1_v, ones_v, bins_v, bins_sh,
             lbl2_v, cr_v, d_v, ridx_v, labsp_v, m2_v, s2_v, g2_v,
             row0_v, row1_v, row2_v, row3_v,
             sem0, sem1, sem2, sem3):
    cid = lax.axis_index("c")
    sid = lax.axis_index("s")
    wid = sid * NC + cid

    iota = lax.iota(jnp.int32, L)
    ones16 = jnp.ones((L,), jnp.float32)
    zeros16 = jnp.zeros((L,), jnp.float32)
    neg16 = jnp.full((L,), -1e30, jnp.float32)

    bufs = [row0_v, row1_v, row2_v, row3_v]
    sems = [sem0, sem1, sem2, sem3]

    base_sc = NT + wid * RS
    for b in range(NBUF):
        pltpu.async_copy(c_hbm.at[base_sc + b], bufs[b], sems[b])

    base2 = wid * CHUNK2
    pltpu.sync_copy(lbl_hbm.at[pl.ds(base2, CHUNK2)], lbl2_v)

    def _fill(j, carry):
        bins_v[pl.ds(j * L, L)] = zeros16
        return carry
    lax.fori_loop(0, KPAD // L, _fill, 0)

    def _fill1(j, carry):
        ones_v[pl.ds(j * L, L)] = ones16
        return carry
    lax.fori_loop(0, SW // L, _fill1, 0)

    base1 = sid * CHUNK1
    for j in range(R1):
        pltpu.sync_copy(lbl_hbm.at[pl.ds(base1 + j * SW, SW)], lbl1_v.at[j])

    @pl.when(sid == 0)
    def _():
        pltpu.sync_copy(bins_v, bins_sh)

    plsc.subcore_barrier()
    for j in range(R1):
        pltpu.sync_copy(ones_v, bins_sh.at[lbl1_v.at[j]], add=True)
    plsc.subcore_barrier()

    pltpu.sync_copy(bins_sh, bins_v)

    for t in range(CHUNK2 // SW):
        pltpu.sync_copy(
            bins_sh.at[lbl2_v.at[pl.ds(t * SW, SW)]],
            cr_v.at[pl.ds(t * SW, SW)],
        )
    pltpu.sync_copy(cr_v, cr_hbm.at[pl.ds(base2, CHUNK2)])

    @pl.when((cid == 0) & (sid == 0))
    def _():
        def _dd(j, a):
            return a + jnp.where(bins_v[pl.ds(j * L, L)] > 0.0, 1.0, 0.0)
        d_v[...] = lax.fori_loop(0, KPAD // L, _dd, zeros16)
        pltpu.sync_copy(d_v, d_hbm)

    def _ri(t, carry):
        ridx_v[pl.ds(t * L, L)] = jnp.full((L,), base_sc + t, jnp.int32)
        return carry
    lax.fori_loop(0, RS, _ri, 0)

    for u in range(RS * L // SW):
        pltpu.sync_copy(
            lbl_hbm.at[ridx_v.at[pl.ds(u * SW, SW)]],
            labsp_v.at[pl.ds(u * SW, SW)],
        )

    ntail = L - K % L

    def _outer(it, carry):
        for b in range(NBUF):
            r = it * NBUF + b
            pltpu.make_async_copy(c_hbm.at[0], bufs[b], sems[b]).wait()
            buf = bufs[b]
            lab16 = labsp_v[pl.ds(r * L, L)]
            tail16 = buf[pl.ds(TAIL, L)]
            m16 = jnp.where(iota < ntail, neg16, tail16)

            def _mx(j, m):
                return jnp.maximum(m, buf[pl.ds(j * L, L)])
            m16 = lax.fori_loop(0, NVF + 1, _mx, m16)

            tcol = TAIL + iota
            s16 = jnp.where(iota < ntail, zeros16, jnp.exp(tail16 - m16))
            g16 = jnp.where((tcol == lab16) & (iota >= ntail), tail16, zeros16)

            def _sm(j, sg):
                s, g = sg
                x = buf[pl.ds(j * L, L)]
                col = j * L + iota
                return (s + jnp.exp(x - m16),
                        g + jnp.where(col == lab16, x, zeros16))
            s16, g16 = lax.fori_loop(0, NVF + 1, _sm, (s16, g16))

            m2_v[pl.ds(r * L, L)] = m16
            s2_v[pl.ds(r * L, L)] = s16
            g2_v[pl.ds(r * L, L)] = g16

            nxt = r + NBUF

            @pl.when(nxt < RS)
            def _():
                pltpu.async_copy(c_hbm.at[base_sc + nxt], bufs[b], sems[b])
        return carry
    lax.fori_loop(0, RS // NBUF, _outer, 0)

    pltpu.sync_copy(m2_v, m_hbm.at[pl.ds(wid * RS * L, RS * L)])
    pltpu.sync_copy(s2_v, s_hbm.at[pl.ds(wid * RS * L, RS * L)])
    pltpu.sync_copy(g2_v, g_hbm.at[pl.ds(wid * RS * L, RS * L)])


_sc_stats = functools.partial(
    pl.kernel,
    mesh=plsc.VectorSubcoreMesh(core_axis_name="c", subcore_axis_name="s"),
    out_type=[
        jax.ShapeDtypeStruct((BATCH,), jnp.float32),
        jax.ShapeDtypeStruct((NSC * L,), jnp.float32),
        jax.ShapeDtypeStruct((NSC * L,), jnp.float32),
        jax.ShapeDtypeStruct((NSC * L,), jnp.float32),
        jax.ShapeDtypeStruct((L,), jnp.float32),
    ],
    scratch_types=[
        pltpu.VMEM((R1, SW), jnp.int32),
        pltpu.VMEM((SW,), jnp.float32),
        pltpu.VMEM((KPAD,), jnp.float32),
        pltpu.VMEM_SHARED((KPAD,), jnp.float32),
        pltpu.VMEM((CHUNK2,), jnp.int32),
        pltpu.VMEM((CHUNK2,), jnp.float32),
        pltpu.VMEM((L,), jnp.float32),
        pltpu.VMEM((RS * L,), jnp.int32),
        pltpu.VMEM((RS * L,), jnp.int32),
        pltpu.VMEM((RS * L,), jnp.float32),
        pltpu.VMEM((RS * L,), jnp.float32),
        pltpu.VMEM((RS * L,), jnp.float32),
        pltpu.VMEM((K,), jnp.float32),
        pltpu.VMEM((K,), jnp.float32),
        pltpu.VMEM((K,), jnp.float32),
        pltpu.VMEM((K,), jnp.float32),
        pltpu.SemaphoreType.DMA,
        pltpu.SemaphoreType.DMA,
        pltpu.SemaphoreType.DMA,
        pltpu.SemaphoreType.DMA,
    ],
)(_sc_body)


def _tc_body(lbl_ref, c_ref, nll_ref):
    cb = c_ref[...]
    m = jnp.max(cb, axis=1, keepdims=True)
    s = jnp.sum(jnp.exp(cb - m), axis=1, keepdims=True)
    lse = m + jnp.log(s)
    onehot = jax.lax.broadcasted_iota(jnp.int32, (BR, K), 1) == lbl_ref[...]
    g = jnp.sum(jnp.where(onehot, cb, 0.0), axis=1, keepdims=True)
    nll_ref[...] = lse - g


def _fin_body(nll_ref, crt_ref, m_ref, s_ref, g_ref, crs_ref, d_ref, loss_ref):
    t1 = jnp.sum(nll_ref[...] / crt_ref[...], keepdims=True)

    m16 = m_ref[...]
    mrow = jnp.max(m16, axis=1, keepdims=True)
    srow = jnp.sum(s_ref[...] * jnp.exp(m16 - mrow), axis=1, keepdims=True)
    grow = jnp.sum(g_ref[...], axis=1, keepdims=True)
    nll_sc = mrow + jnp.log(srow) - grow
    t2 = jnp.sum(nll_sc / crs_ref[...], keepdims=True)

    den = jnp.sum(d_ref[...], keepdims=True)
    loss_ref[...] = (t1 + t2) / den


def kernel(c, pseudo_label):
    lbl = pseudo_label.astype(jnp.int32)
    cr, m_sc, s_sc, g_sc, dv = _sc_stats(lbl, c)

    nll_tc = pl.pallas_call(
        _tc_body,
        grid=(NBT,),
        in_specs=[
            pl.BlockSpec((BR, 1), lambda k: (k, 0)),
            pl.BlockSpec((BR, K), lambda k: (k, 0)),
        ],
        out_specs=pl.BlockSpec((BR, 1), lambda k: (k, 0)),
        out_shape=jax.ShapeDtypeStruct((NT, 1), jnp.float32),
    )(lbl[:NT].reshape(NT, 1), c)

    loss = pl.pallas_call(
        _fin_body,
        in_specs=[
            pl.BlockSpec((NT, 1), lambda: (0, 0)),
            pl.BlockSpec((NT, 1), lambda: (0, 0)),
            pl.BlockSpec((NSC, L), lambda: (0, 0)),
            pl.BlockSpec((NSC, L), lambda: (0, 0)),
            pl.BlockSpec((NSC, L), lambda: (0, 0)),
            pl.BlockSpec((NSC, 1), lambda: (0, 0)),
            pl.BlockSpec((1, L), lambda: (0, 0)),
        ],
        out_specs=pl.BlockSpec((1, 1), lambda: (0, 0)),
        out_shape=jax.ShapeDtypeStruct((1, 1), jnp.float32),
    )(
        nll_tc,
        cr[:NT].reshape(NT, 1),
        m_sc.reshape(NSC, L),
        s_sc.reshape(NSC, L),
        g_sc.reshape(NSC, L),
        cr[NT:].reshape(NSC, 1),
        dv.reshape(1, L),
    )
    return loss[0, 0]


# speedup vs baseline: 1.3094x; 1.3094x over previous
import functools

import jax
import jax.numpy as jnp
from jax import lax
from jax.experimental import pallas as pl
from jax.experimental.pallas import tpu as pltpu
from jax.experimental.pallas import tpu_sc as plsc

BATCH = 16384
K = 1000
BR = 512

L = 16
NC = 2
NS = 16
NW = NC * NS
CHUNK1 = BATCH // NS
CHUNK2 = BATCH // NW
KPAD = 1024
SW = 128
R1 = CHUNK1 // SW

RS = 160
NSC = NW * RS
NT = BATCH - NSC
NBUF = 4
NVF = (K - L) // L
TAIL = K - L
NBT = NT // BR


def _sc_body(lbl_hbm, c_hbm, cr_hbm, m_hbm, s_hbm, g_hbm, d_hbm,
             lbl1_v, ones_v, bins_v, bins_sh,
             lbl2_v, cr_v, d_v, ridx_v, labsp_v, m2_v, s2_v, g2_v,
             row0_v, row1_v, row2_v, row3_v,
             sem0, sem1, sem2, sem3):
    cid = lax.axis_index("c")
    sid = lax.axis_index("s")
    wid = sid * NC + cid

    iota = lax.iota(jnp.int32, L)
    ones16 = jnp.ones((L,), jnp.float32)
    zeros16 = jnp.zeros((L,), jnp.float32)
    neg16 = jnp.full((L,), -1e30, jnp.float32)

    bufs = [row0_v, row1_v, row2_v, row3_v]
    sems = [sem0, sem1, sem2, sem3]

    base_sc = NT + wid * RS
    for b in range(NBUF):
        pltpu.async_copy(c_hbm.at[base_sc + b], bufs[b], sems[b])

    base2 = wid * CHUNK2
    pltpu.sync_copy(lbl_hbm.at[pl.ds(base2, CHUNK2)], lbl2_v)

    def _fill(j, carry):
        bins_v[pl.ds(j * L, L)] = zeros16
        return carry
    lax.fori_loop(0, KPAD // L, _fill, 0)

    def _fill1(j, carry):
        ones_v[pl.ds(j * L, L)] = ones16
        return carry
    lax.fori_loop(0, SW // L, _fill1, 0)

    base1 = sid * CHUNK1
    for j in range(R1):
        pltpu.sync_copy(lbl_hbm.at[pl.ds(base1 + j * SW, SW)], lbl1_v.at[j])

    @pl.when(sid == 0)
    def _():
        pltpu.sync_copy(bins_v, bins_sh)

    plsc.subcore_barrier()
    for j in range(R1):
        pltpu.sync_copy(ones_v, bins_sh.at[lbl1_v.at[j]], add=True)
    plsc.subcore_barrier()

    pltpu.sync_copy(bins_sh, bins_v)

    for t in range(CHUNK2 // SW):
        pltpu.sync_copy(
            bins_sh.at[lbl2_v.at[pl.ds(t * SW, SW)]],
            cr_v.at[pl.ds(t * SW, SW)],
        )
    pltpu.sync_copy(cr_v, cr_hbm.at[pl.ds(base2, CHUNK2)])

    @pl.when((cid == 0) & (sid == 0))
    def _():
        def _dd(j, a):
            return a + jnp.where(bins_v[pl.ds(j * L, L)] > 0.0, 1.0, 0.0)
        d_v[...] = lax.fori_loop(0, KPAD // L, _dd, zeros16)
        pltpu.sync_copy(d_v, d_hbm)

    def _ri(t, carry):
        ridx_v[pl.ds(t * L, L)] = jnp.full((L,), base_sc + t, jnp.int32)
        return carry
    lax.fori_loop(0, RS, _ri, 0)

    for u in range(RS * L // SW):
        pltpu.sync_copy(
            lbl_hbm.at[ridx_v.at[pl.ds(u * SW, SW)]],
            labsp_v.at[pl.ds(u * SW, SW)],
        )

    ntail = L - K % L

    def _outer(it, carry):
        for b in range(NBUF):
            r = it * NBUF + b
            pltpu.make_async_copy(c_hbm.at[0], bufs[b], sems[b]).wait()
            buf = bufs[b]
            lab16 = labsp_v[pl.ds(r * L, L)]

            tail16 = buf[pl.ds(TAIL, L)]
            macc = [jnp.where(iota < ntail, neg16, tail16), neg16, neg16, neg16]
            for j in range(NVF + 1):
                macc[j % 4] = jnp.maximum(macc[j % 4], buf[pl.ds(j * L, L)])
            m16 = jnp.maximum(jnp.maximum(macc[0], macc[1]),
                              jnp.maximum(macc[2], macc[3]))

            tcol = TAIL + iota
            sacc = [jnp.where(iota < ntail, zeros16, jnp.exp(tail16 - m16)),
                    zeros16, zeros16, zeros16]
            gacc = [jnp.where((tcol == lab16) & (iota >= ntail), tail16, zeros16),
                    zeros16, zeros16, zeros16]
            for j in range(NVF + 1):
                x = buf[pl.ds(j * L, L)]
                sacc[j % 4] = sacc[j % 4] + jnp.exp(x - m16)
                gacc[j % 4] = gacc[j % 4] + jnp.where(j * L + iota == lab16,
                                                      x, zeros16)
            s16 = (sacc[0] + sacc[1]) + (sacc[2] + sacc[3])
            g16 = (gacc[0] + gacc[1]) + (gacc[2] + gacc[3])

            m2_v[pl.ds(r * L, L)] = m16
            s2_v[pl.ds(r * L, L)] = s16
            g2_v[pl.ds(r * L, L)] = g16

            nxt = r + NBUF

            @pl.when(nxt < RS)
            def _():
                pltpu.async_copy(c_hbm.at[base_sc + nxt], bufs[b], sems[b])
        return carry
    lax.fori_loop(0, RS // NBUF, _outer, 0)

    pltpu.sync_copy(m2_v, m_hbm.at[pl.ds(wid * RS * L, RS * L)])
    pltpu.sync_copy(s2_v, s_hbm.at[pl.ds(wid * RS * L, RS * L)])
    pltpu.sync_copy(g2_v, g_hbm.at[pl.ds(wid * RS * L, RS * L)])


_sc_stats = functools.partial(
    pl.kernel,
    mesh=plsc.VectorSubcoreMesh(core_axis_name="c", subcore_axis_name="s"),
    out_type=[
        jax.ShapeDtypeStruct((BATCH,), jnp.float32),
        jax.ShapeDtypeStruct((NSC * L,), jnp.float32),
        jax.ShapeDtypeStruct((NSC * L,), jnp.float32),
        jax.ShapeDtypeStruct((NSC * L,), jnp.float32),
        jax.ShapeDtypeStruct((L,), jnp.float32),
    ],
    scratch_types=[
        pltpu.VMEM((R1, SW), jnp.int32),
        pltpu.VMEM((SW,), jnp.float32),
        pltpu.VMEM((KPAD,), jnp.float32),
        pltpu.VMEM_SHARED((KPAD,), jnp.float32),
        pltpu.VMEM((CHUNK2,), jnp.int32),
        pltpu.VMEM((CHUNK2,), jnp.float32),
        pltpu.VMEM((L,), jnp.float32),
        pltpu.VMEM((RS * L,), jnp.int32),
        pltpu.VMEM((RS * L,), jnp.int32),
        pltpu.VMEM((RS * L,), jnp.float32),
        pltpu.VMEM((RS * L,), jnp.float32),
        pltpu.VMEM((RS * L,), jnp.float32),
        pltpu.VMEM((K,), jnp.float32),
        pltpu.VMEM((K,), jnp.float32),
        pltpu.VMEM((K,), jnp.float32),
        pltpu.VMEM((K,), jnp.float32),
        pltpu.SemaphoreType.DMA,
        pltpu.SemaphoreType.DMA,
        pltpu.SemaphoreType.DMA,
        pltpu.SemaphoreType.DMA,
    ],
)(_sc_body)


def _tc_body(lbl_ref, c_ref, nll_ref):
    cb = c_ref[...]
    m = jnp.max(cb, axis=1, keepdims=True)
    s = jnp.sum(jnp.exp(cb - m), axis=1, keepdims=True)
    lse = m + jnp.log(s)
    onehot = jax.lax.broadcasted_iota(jnp.int32, (BR, K), 1) == lbl_ref[...]
    g = jnp.sum(jnp.where(onehot, cb, 0.0), axis=1, keepdims=True)
    nll_ref[...] = lse - g


def _fin_body(nll_ref, crt_ref, m_ref, s_ref, g_ref, crs_ref, d_ref, loss_ref):
    t1 = jnp.sum(nll_ref[...] / crt_ref[...], keepdims=True)

    m16 = m_ref[...]
    mrow = jnp.max(m16, axis=1, keepdims=True)
    srow = jnp.sum(s_ref[...] * jnp.exp(m16 - mrow), axis=1, keepdims=True)
    grow = jnp.sum(g_ref[...], axis=1, keepdims=True)
    nll_sc = mrow + jnp.log(srow) - grow
    t2 = jnp.sum(nll_sc / crs_ref[...], keepdims=True)

    den = jnp.sum(d_ref[...], keepdims=True)
    loss_ref[...] = (t1 + t2) / den


def kernel(c, pseudo_label):
    lbl = pseudo_label.astype(jnp.int32)
    cr, m_sc, s_sc, g_sc, dv = _sc_stats(lbl, c)

    nll_tc = pl.pallas_call(
        _tc_body,
        grid=(NBT,),
        in_specs=[
            pl.BlockSpec((BR, 1), lambda k: (k, 0)),
            pl.BlockSpec((BR, K), lambda k: (k, 0)),
        ],
        out_specs=pl.BlockSpec((BR, 1), lambda k: (k, 0)),
        out_shape=jax.ShapeDtypeStruct((NT, 1), jnp.float32),
    )(lbl[:NT].reshape(NT, 1), c)

    loss = pl.pallas_call(
        _fin_body,
        in_specs=[
            pl.BlockSpec((NT, 1), lambda: (0, 0)),
            pl.BlockSpec((NT, 1), lambda: (0, 0)),
            pl.BlockSpec((NSC, L), lambda: (0, 0)),
            pl.BlockSpec((NSC, L), lambda: (0, 0)),
            pl.BlockSpec((NSC, L), lambda: (0, 0)),
            pl.BlockSpec((NSC, 1), lambda: (0, 0)),
            pl.BlockSpec((1, L), lambda: (0, 0)),
        ],
        out_specs=pl.BlockSpec((1, 1), lambda: (0, 0)),
        out_shape=jax.ShapeDtypeStruct((1, 1), jnp.float32),
    )(
        nll_tc,
        cr[:NT].reshape(NT, 1),
        m_sc.reshape(NSC, L),
        s_sc.reshape(NSC, L),
        g_sc.reshape(NSC, L),
        cr[NT:].reshape(NSC, 1),
        dv.reshape(1, L),
    )
    return loss[0, 0]


# speedup vs baseline: 1.3143x; 1.0037x over previous
import functools

import jax
import jax.numpy as jnp
from jax import lax
from jax.experimental import pallas as pl
from jax.experimental.pallas import tpu as pltpu
from jax.experimental.pallas import tpu_sc as plsc

BATCH = 16384
K = 1000
BR = 512

L = 16
NC = 2
NS = 16
NW = NC * NS
CHUNK1 = BATCH // NS
CHUNK2 = BATCH // NW
KPAD = 1024
SW = 128
R1 = CHUNK1 // SW

RS = 160
NSC = NW * RS
NT = BATCH - NSC
NBUF = 4
NVF = (K - L) // L
TAIL = K - L
NBT = NT // BR


def _sc_body(lbl_hbm, c_hbm, cr_hbm, m_hbm, s_hbm, g_hbm, d_hbm,
             lbl1_v, ones_v, bins_v, bins_sh,
             lbl2_v, cr_v, d_v, ridx_v, labsp_v, m2_v, s2_v, g2_v,
             row0_v, row1_v, row2_v, row3_v,
             sem0, sem1, sem2, sem3):
    cid = lax.axis_index("c")
    sid = lax.axis_index("s")
    wid = sid * NC + cid

    iota = lax.iota(jnp.int32, L)
    ones16 = jnp.ones((L,), jnp.float32)
    zeros16 = jnp.zeros((L,), jnp.float32)
    neg16 = jnp.full((L,), -1e30, jnp.float32)

    bufs = [row0_v, row1_v, row2_v, row3_v]
    sems = [sem0, sem1, sem2, sem3]

    base_sc = NT + wid * RS
    for b in range(NBUF):
        pltpu.async_copy(c_hbm.at[base_sc + b], bufs[b], sems[b])

    base2 = wid * CHUNK2
    pltpu.sync_copy(lbl_hbm.at[pl.ds(base2, CHUNK2)], lbl2_v)

    def _fill(j, carry):
        bins_v[pl.ds(j * L, L)] = zeros16
        return carry
    lax.fori_loop(0, KPAD // L, _fill, 0)

    def _fill1(j, carry):
        ones_v[pl.ds(j * L, L)] = ones16
        return carry
    lax.fori_loop(0, SW // L, _fill1, 0)

    base1 = sid * CHUNK1
    for j in range(R1):
        pltpu.sync_copy(lbl_hbm.at[pl.ds(base1 + j * SW, SW)], lbl1_v.at[j])

    @pl.when(sid == 0)
    def _():
        pltpu.sync_copy(bins_v, bins_sh)

    plsc.subcore_barrier()
    for j in range(R1):
        pltpu.sync_copy(ones_v, bins_sh.at[lbl1_v.at[j]], add=True)
    plsc.subcore_barrier()

    pltpu.sync_copy(bins_sh, bins_v)

    for t in range(CHUNK2 // SW):
        pltpu.sync_copy(
            bins_sh.at[lbl2_v.at[pl.ds(t * SW, SW)]],
            cr_v.at[pl.ds(t * SW, SW)],
        )
    pltpu.sync_copy(cr_v, cr_hbm.at[pl.ds(base2, CHUNK2)])

    @pl.when((cid == 0) & (sid == 0))
    def _():
        def _dd(j, a):
            return a + jnp.where(bins_v[pl.ds(j * L, L)] > 0.0, 1.0, 0.0)
        d_v[...] = lax.fori_loop(0, KPAD // L, _dd, zeros16)
        pltpu.sync_copy(d_v, d_hbm)

    def _ri(t, carry):
        ridx_v[pl.ds(t * L, L)] = jnp.full((L,), base_sc + t, jnp.int32)
        return carry
    lax.fori_loop(0, RS, _ri, 0)

    for u in range(RS * L // SW):
        pltpu.sync_copy(
            lbl_hbm.at[ridx_v.at[pl.ds(u * SW, SW)]],
            labsp_v.at[pl.ds(u * SW, SW)],
        )

    ntail = L - K % L

    def _outer(it, carry):
        for b in range(NBUF):
            r = it * NBUF + b
            pltpu.make_async_copy(c_hbm.at[0], bufs[b], sems[b]).wait()
            buf = bufs[b]
            lab16 = labsp_v[pl.ds(r * L, L)]

            tail16 = buf[pl.ds(TAIL, L)]
            macc = [jnp.where(iota < ntail, neg16, tail16), neg16, neg16, neg16]
            for j in range(NVF + 1):
                macc[j % 4] = jnp.maximum(macc[j % 4], buf[pl.ds(j * L, L)])
            m16 = jnp.maximum(jnp.maximum(macc[0], macc[1]),
                              jnp.maximum(macc[2], macc[3]))

            tcol = TAIL + iota
            sacc = [jnp.where(iota < ntail, zeros16, jnp.exp(tail16 - m16)),
                    zeros16, zeros16, zeros16]
            gacc = [jnp.where((tcol == lab16) & (iota >= ntail), tail16, zeros16),
                    zeros16, zeros16, zeros16]
            for j in range(NVF + 1):
                x = buf[pl.ds(j * L, L)]
                sacc[j % 4] = sacc[j % 4] + jnp.exp(x - m16)
                gacc[j % 4] = gacc[j % 4] + jnp.where(j * L + iota == lab16,
                                                      x, zeros16)
            s16 = (sacc[0] + sacc[1]) + (sacc[2] + sacc[3])
            g16 = (gacc[0] + gacc[1]) + (gacc[2] + gacc[3])

            m2_v[pl.ds(r * L, L)] = m16
            s2_v[pl.ds(r * L, L)] = s16
            g2_v[pl.ds(r * L, L)] = g16

            nxt = r + NBUF

            @pl.when(nxt < RS)
            def _():
                pltpu.async_copy(c_hbm.at[base_sc + nxt], bufs[b], sems[b])
        return carry
    lax.fori_loop(0, RS // NBUF, _outer, 0)

    pltpu.sync_copy(m2_v, m_hbm.at[pl.ds(wid * RS * L, RS * L)])
    pltpu.sync_copy(s2_v, s_hbm.at[pl.ds(wid * RS * L, RS * L)])
    pltpu.sync_copy(g2_v, g_hbm.at[pl.ds(wid * RS * L, RS * L)])


_sc_stats = functools.partial(
    pl.kernel,
    mesh=plsc.VectorSubcoreMesh(core_axis_name="c", subcore_axis_name="s"),
    compiler_params=pltpu.CompilerParams(use_tc_tiling_on_sc=True),
    out_type=[
        jax.ShapeDtypeStruct((BATCH,), jnp.float32),
        jax.ShapeDtypeStruct((NSC * L,), jnp.float32),
        jax.ShapeDtypeStruct((NSC * L,), jnp.float32),
        jax.ShapeDtypeStruct((NSC * L,), jnp.float32),
        jax.ShapeDtypeStruct((L,), jnp.float32),
    ],
    scratch_types=[
        pltpu.VMEM((R1, SW), jnp.int32),
        pltpu.VMEM((SW,), jnp.float32),
        pltpu.VMEM((KPAD,), jnp.float32),
        pltpu.VMEM_SHARED((KPAD,), jnp.float32),
        pltpu.VMEM((CHUNK2,), jnp.int32),
        pltpu.VMEM((CHUNK2,), jnp.float32),
        pltpu.VMEM((L,), jnp.float32),
        pltpu.VMEM((RS * L,), jnp.int32),
        pltpu.VMEM((RS * L,), jnp.int32),
        pltpu.VMEM((RS * L,), jnp.float32),
        pltpu.VMEM((RS * L,), jnp.float32),
        pltpu.VMEM((RS * L,), jnp.float32),
        pltpu.VMEM((K,), jnp.float32),
        pltpu.VMEM((K,), jnp.float32),
        pltpu.VMEM((K,), jnp.float32),
        pltpu.VMEM((K,), jnp.float32),
        pltpu.SemaphoreType.DMA,
        pltpu.SemaphoreType.DMA,
        pltpu.SemaphoreType.DMA,
        pltpu.SemaphoreType.DMA,
    ],
)(_sc_body)


def _tc_body(lbl_ref, c_ref, nll_ref):
    cb = c_ref[...]
    m = jnp.max(cb, axis=1, keepdims=True)
    s = jnp.sum(jnp.exp(cb - m), axis=1, keepdims=True)
    lse = m + jnp.log(s)
    onehot = jax.lax.broadcasted_iota(jnp.int32, (BR, K), 1) == lbl_ref[...]
    g = jnp.sum(jnp.where(onehot, cb, 0.0), axis=1, keepdims=True)
    nll_ref[...] = lse - g


def _fin_body(nll_ref, crt_ref, m_ref, s_ref, g_ref, crs_ref, d_ref, loss_ref):
    t1 = jnp.sum(nll_ref[...] / crt_ref[...], keepdims=True)

    m16 = m_ref[...]
    mrow = jnp.max(m16, axis=1, keepdims=True)
    srow = jnp.sum(s_ref[...] * jnp.exp(m16 - mrow), axis=1, keepdims=True)
    grow = jnp.sum(g_ref[...], axis=1, keepdims=True)
    nll_sc = mrow + jnp.log(srow) - grow
    t2 = jnp.sum(nll_sc / crs_ref[...], keepdims=True)

    den = jnp.sum(d_ref[...], keepdims=True)
    loss_ref[...] = (t1 + t2) / den


def kernel(c, pseudo_label):
    lbl = pseudo_label.astype(jnp.int32)
    cr, m_sc, s_sc, g_sc, dv = _sc_stats(lbl, c)

    nll_tc = pl.pallas_call(
        _tc_body,
        grid=(NBT,),
        in_specs=[
            pl.BlockSpec((BR, 1), lambda k: (k, 0)),
            pl.BlockSpec((BR, K), lambda k: (k, 0)),
        ],
        out_specs=pl.BlockSpec((BR, 1), lambda k: (k, 0)),
        out_shape=jax.ShapeDtypeStruct((NT, 1), jnp.float32),
    )(lbl[:NT].reshape(NT, 1), c)

    loss = pl.pallas_call(
        _fin_body,
        in_specs=[
            pl.BlockSpec((NT, 1), lambda: (0, 0)),
            pl.BlockSpec((NT, 1), lambda: (0, 0)),
            pl.BlockSpec((NSC, L), lambda: (0, 0)),
            pl.BlockSpec((NSC, L), lambda: (0, 0)),
            pl.BlockSpec((NSC, L), lambda: (0, 0)),
            pl.BlockSpec((NSC, 1), lambda: (0, 0)),
            pl.BlockSpec((1, L), lambda: (0, 0)),
        ],
        out_specs=pl.BlockSpec((1, 1), lambda: (0, 0)),
        out_shape=jax.ShapeDtypeStruct((1, 1), jnp.float32),
    )(
        nll_tc,
        cr[:NT].reshape(NT, 1),
        m_sc.reshape(NSC, L),
        s_sc.reshape(NSC, L),
        g_sc.reshape(NSC, L),
        cr[NT:].reshape(NSC, 1),
        dv.reshape(1, L),
    )
    return loss[0, 0]


# speedup vs baseline: 3.5331x; 2.6882x over previous
import functools

import jax
import jax.numpy as jnp
from jax import lax
from jax.experimental import pallas as pl
from jax.experimental.pallas import tpu as pltpu
from jax.experimental.pallas import tpu_sc as plsc

BATCH = 16384
K = 1000
BR = 512
NB = BATCH // BR

L = 16
NC = 2
NS = 16
NW = NC * NS
CHUNK1 = BATCH // NS
CHUNK2 = BATCH // NW
KPAD = 1024
SW = 128
R1 = CHUNK1 // SW


def _sc_body(lbl_hbm, cr_hbm, d_hbm,
             lbl1_v, ones_v, bins_v, bins_sh,
             lbl2_v, cr_v, d_v):
    cid = lax.axis_index("c")
    sid = lax.axis_index("s")
    wid = sid * NC + cid

    ones16 = jnp.ones((L,), jnp.float32)
    zeros16 = jnp.zeros((L,), jnp.float32)

    base2 = wid * CHUNK2
    pltpu.sync_copy(lbl_hbm.at[pl.ds(base2, CHUNK2)], lbl2_v)

    def _fill(j, carry):
        bins_v[pl.ds(j * L, L)] = zeros16
        return carry
    lax.fori_loop(0, KPAD // L, _fill, 0)

    def _fill1(j, carry):
        ones_v[pl.ds(j * L, L)] = ones16
        return carry
    lax.fori_loop(0, SW // L, _fill1, 0)

    base1 = sid * CHUNK1
    for j in range(R1):
        pltpu.sync_copy(lbl_hbm.at[pl.ds(base1 + j * SW, SW)], lbl1_v.at[j])

    @pl.when(sid == 0)
    def _():
        pltpu.sync_copy(bins_v, bins_sh)

    plsc.subcore_barrier()
    for j in range(R1):
        pltpu.sync_copy(ones_v, bins_sh.at[lbl1_v.at[j]], add=True)
    plsc.subcore_barrier()

    pltpu.sync_copy(bins_sh, bins_v)

    for t in range(CHUNK2 // SW):
        pltpu.sync_copy(
            bins_sh.at[lbl2_v.at[pl.ds(t * SW, SW)]],
            cr_v.at[pl.ds(t * SW, SW)],
        )
    pltpu.sync_copy(cr_v, cr_hbm.at[pl.ds(base2, CHUNK2)])

    @pl.when((cid == 0) & (sid == 0))
    def _():
        def _dd(j, a):
            return a + jnp.where(bins_v[pl.ds(j * L, L)] > 0.0, 1.0, 0.0)
        d_v[...] = lax.fori_loop(0, KPAD // L, _dd, zeros16)
        pltpu.sync_copy(d_v, d_hbm)


_sc_stats = functools.partial(
    pl.kernel,
    mesh=plsc.VectorSubcoreMesh(core_axis_name="c", subcore_axis_name="s"),
    out_type=[
        jax.ShapeDtypeStruct((BATCH,), jnp.float32),
        jax.ShapeDtypeStruct((L,), jnp.float32),
    ],
    scratch_types=[
        pltpu.VMEM((R1, SW), jnp.int32),
        pltpu.VMEM((SW,), jnp.float32),
        pltpu.VMEM((KPAD,), jnp.float32),
        pltpu.VMEM_SHARED((KPAD,), jnp.float32),
        pltpu.VMEM((CHUNK2,), jnp.int32),
        pltpu.VMEM((CHUNK2,), jnp.float32),
        pltpu.VMEM((L,), jnp.float32),
    ],
)(_sc_body)


def _tc_body(lbl_ref, cr_ref, d_ref, ct_ref, loss_ref, acc_s):
    k = pl.program_id(0)

    @pl.when(k == 0)
    def _():
        acc_s[...] = jnp.zeros_like(acc_s)

    cb = ct_ref[...]
    m = jnp.max(cb, axis=0, keepdims=True)
    s = jnp.sum(jnp.exp(cb - m), axis=0, keepdims=True)
    lse = m + jnp.log(s)

    onehot = jax.lax.broadcasted_iota(jnp.int32, (K, BR), 0) == lbl_ref[...]
    g = jnp.sum(jnp.where(onehot, cb, 0.0), axis=0, keepdims=True)
    val = (lse - g) / cr_ref[...]
    acc_s[...] += jnp.sum(val, keepdims=True)

    @pl.when(k == NB - 1)
    def _():
        loss_ref[...] = acc_s[...] / jnp.sum(d_ref[...], keepdims=True)


def kernel(c, pseudo_label):
    lbl = pseudo_label.astype(jnp.int32)
    cr, dv = _sc_stats(lbl)

    out = pl.pallas_call(
        _tc_body,
        grid=(NB,),
        in_specs=[
            pl.BlockSpec((1, BR), lambda k: (0, k)),
            pl.BlockSpec((1, BR), lambda k: (0, k)),
            pl.BlockSpec((1, L), lambda k: (0, 0)),
            pl.BlockSpec((K, BR), lambda k: (0, k)),
        ],
        out_specs=pl.BlockSpec((1, 1), lambda k: (0, 0)),
        out_shape=jax.ShapeDtypeStruct((1, 1), jnp.float32),
        scratch_shapes=[pltpu.VMEM((1, 1), jnp.float32)],
    )(lbl.reshape(1, BATCH), cr.reshape(1, BATCH), dv.reshape(1, L), c.T)
    return out[0, 0]


# speedup vs baseline: 4.2334x; 1.1982x over previous
import functools

import jax
import jax.numpy as jnp
from jax import lax
from jax.experimental import pallas as pl
from jax.experimental.pallas import tpu as pltpu
from jax.experimental.pallas import tpu_sc as plsc

BATCH = 16384
K = 1000
BR = 512
NB = BATCH // BR

L = 16
NC = 2
NS = 16
NW = NC * NS
CHUNK1 = BATCH // NS
CHUNK2 = BATCH // NW
KPAD = 1024
SW = 128
R1 = CHUNK1 // SW


def _sc_body(lbl_hbm, cr_hbm, d_hbm,
             lbl1_v, ones_v, bins_v, bins_sh,
             lbl2_v, cr_v, d_v):
    cid = lax.axis_index("c")
    sid = lax.axis_index("s")
    wid = sid * NC + cid

    ones16 = jnp.ones((L,), jnp.float32)
    zeros16 = jnp.zeros((L,), jnp.float32)

    base2 = wid * CHUNK2
    pltpu.sync_copy(lbl_hbm.at[pl.ds(base2, CHUNK2)], lbl2_v)

    def _fill(j, carry):
        bins_v[pl.ds(j * L, L)] = zeros16
        return carry
    lax.fori_loop(0, KPAD // L, _fill, 0)

    def _fill1(j, carry):
        ones_v[pl.ds(j * L, L)] = ones16
        return carry
    lax.fori_loop(0, SW // L, _fill1, 0)

    base1 = sid * CHUNK1
    for j in range(R1):
        pltpu.sync_copy(lbl_hbm.at[pl.ds(base1 + j * SW, SW)], lbl1_v.at[j])

    @pl.when(sid == 0)
    def _():
        pltpu.sync_copy(bins_v, bins_sh)

    plsc.subcore_barrier()
    for j in range(R1):
        pltpu.sync_copy(ones_v, bins_sh.at[lbl1_v.at[j]], add=True)
    plsc.subcore_barrier()

    pltpu.sync_copy(bins_sh, bins_v)

    for t in range(CHUNK2 // SW):
        pltpu.sync_copy(
            bins_sh.at[lbl2_v.at[pl.ds(t * SW, SW)]],
            cr_v.at[pl.ds(t * SW, SW)],
        )
    pltpu.sync_copy(cr_v, cr_hbm.at[pl.ds(base2, CHUNK2)])

    @pl.when((cid == 0) & (sid == 0))
    def _():
        def _dd(j, a):
            return a + jnp.where(bins_v[pl.ds(j * L, L)] > 0.0, 1.0, 0.0)
        d_v[...] = lax.fori_loop(0, KPAD // L, _dd, zeros16)
        pltpu.sync_copy(d_v, d_hbm)


_sc_stats = functools.partial(
    pl.kernel,
    mesh=plsc.VectorSubcoreMesh(core_axis_name="c", subcore_axis_name="s"),
    out_type=[
        jax.ShapeDtypeStruct((BATCH,), jnp.float32),
        jax.ShapeDtypeStruct((L,), jnp.float32),
    ],
    scratch_types=[
        pltpu.VMEM((R1, SW), jnp.int32),
        pltpu.VMEM((SW,), jnp.float32),
        pltpu.VMEM((KPAD,), jnp.float32),
        pltpu.VMEM_SHARED((KPAD,), jnp.float32),
        pltpu.VMEM((CHUNK2,), jnp.int32),
        pltpu.VMEM((CHUNK2,), jnp.float32),
        pltpu.VMEM((L,), jnp.float32),
    ],
)(_sc_body)


def _tc_body(lbl_ref, ct_ref, nll_ref):
    cb = ct_ref[...]
    m = jnp.max(cb, axis=0, keepdims=True)
    s = jnp.sum(jnp.exp(cb - m), axis=0, keepdims=True)
    lse = m + jnp.log(s)

    onehot = jax.lax.broadcasted_iota(jnp.int32, (K, BR), 0) == lbl_ref[...]
    g = jnp.sum(jnp.where(onehot, cb, 0.0), axis=0, keepdims=True)
    nll_ref[...] = lse - g


def _fin_body(nll_ref, cr_ref, d_ref, loss_ref):
    t = jnp.sum(nll_ref[...] / cr_ref[...], keepdims=True)
    loss_ref[...] = t / jnp.sum(d_ref[...], keepdims=True)


def kernel(c, pseudo_label):
    lbl = pseudo_label.astype(jnp.int32)
    cr, dv = _sc_stats(lbl)

    nll = pl.pallas_call(
        _tc_body,
        grid=(NB,),
        in_specs=[
            pl.BlockSpec((1, BR), lambda k: (0, k)),
            pl.BlockSpec((K, BR), lambda k: (0, k)),
        ],
        out_specs=pl.BlockSpec((1, BR), lambda k: (0, k)),
        out_shape=jax.ShapeDtypeStruct((1, BATCH), jnp.float32),
    )(lbl.reshape(1, BATCH), c.T)

    loss = pl.pallas_call(
        _fin_body,
        in_specs=[
            pl.BlockSpec((1, BATCH), lambda: (0, 0)),
            pl.BlockSpec((1, BATCH), lambda: (0, 0)),
            pl.BlockSpec((1, L), lambda: (0, 0)),
        ],
        out_specs=pl.BlockSpec((1, 1), lambda: (0, 0)),
        out_shape=jax.ShapeDtypeStruct((1, 1), jnp.float32),
    )(nll, cr.reshape(1, BATCH), dv.reshape(1, L))
    return loss[0, 0]
